# baseline (speedup 1.0000x reference)
import functools
import math

import jax
import jax.numpy as jnp
from jax import lax
from jax.experimental import pallas as pl
from jax.experimental.pallas import tpu as pltpu
from jax.experimental.pallas import tpu_sc as plsc

BSZ, NODES, EDGES, D = 8, 512, 2048, 512
N_TOTAL = BSZ * NODES
E_TOTAL = BSZ * EDGES
G_TOTAL = 2 * E_TOTAL

_PROJ_BLK = 512


def _proj_body(x_ref, w_ref, out_ref):
    out_ref[...] = jnp.dot(x_ref[...], w_ref[0],
                           preferred_element_type=jnp.float32)


def _project_nodes(x, w_stack):
    nblk = N_TOTAL // _PROJ_BLK
    return pl.pallas_call(
        _proj_body,
        grid=(2 * nblk,),
        in_specs=[
            pl.BlockSpec((_PROJ_BLK, D), lambda j: (j % nblk, 0)),
            pl.BlockSpec((1, D, D), lambda j: (j // nblk, 0, 0)),
        ],
        out_specs=pl.BlockSpec((_PROJ_BLK, D), lambda j: (j, 0)),
        out_shape=jax.ShapeDtypeStruct((2 * N_TOTAL, D), jnp.float32),
    )(x, w_stack)


_NC, _NS = 2, 16
_NW = _NC * _NS
_PER_TILE = G_TOTAL // _NW
_CHUNK = 128
_NCHUNK = _PER_TILE // _CHUNK

@functools.cache
def _sc_gather_kernel():
    mesh = plsc.VectorSubcoreMesh(core_axis_name="c", subcore_axis_name="s",
                                  num_cores=_NC, num_subcores=_NS)

    @functools.partial(
        pl.kernel,
        out_type=jax.ShapeDtypeStruct((G_TOTAL, D), jnp.float32),
        mesh=mesh,
        scratch_types=[
            pltpu.VMEM((_CHUNK,), jnp.int32),
            pltpu.VMEM((_CHUNK, D), jnp.float32),
            pltpu.SemaphoreType.DMA,
        ],
    )
    def body(tbl_hbm, idx_hbm, out_hbm, idx_v, rows_v, sem):
        wid = lax.axis_index("s") * _NC + lax.axis_index("c")
        base = wid * _PER_TILE
        for c in range(_NCHUNK):
            off = base + c * _CHUNK
            pltpu.sync_copy(idx_hbm.at[pl.ds(off, _CHUNK)], idx_v)
            pltpu.async_copy(tbl_hbm.at[idx_v], rows_v, sem).wait()
            pltpu.sync_copy(rows_v, out_hbm.at[pl.ds(off, _CHUNK)])

    return body


def _sc_gather(tbl, idx):
    return _sc_gather_kernel()(tbl, idx)


_MLP_BLK = 1024
_INV_SQRT2 = 1.0 / math.sqrt(2.0)


def _rel_body(r_ref, w1r_ref, out_ref):
    out_ref[...] = jnp.dot(r_ref[...], w1r_ref[...],
                           preferred_element_type=jnp.float32)


def _rel_proj(edge_rep, w1r):
    eblk = E_TOTAL // _MLP_BLK
    return pl.pallas_call(
        _rel_body,
        grid=(eblk,),
        in_specs=[
            pl.BlockSpec((_MLP_BLK, D), lambda j: (j, 0)),
            pl.BlockSpec((D, D), lambda j: (0, 0)),
        ],
        out_specs=pl.BlockSpec((_MLP_BLK, D), lambda j: (j, 0)),
        out_shape=jax.ShapeDtypeStruct((E_TOTAL, D), jnp.float32),
    )(edge_rep, w1r)


def _mlp_body(gh_ref, gt_ref, m_ref, b1_ref, gamma_ref, beta_ref,
              w2_ref, b2_ref, out_ref):
    h1 = m_ref[...] + gh_ref[...] + gt_ref[...] + b1_ref[...]
    mu = jnp.mean(h1, axis=-1, keepdims=True)
    var = jnp.mean((h1 - mu) ** 2, axis=-1, keepdims=True)
    h1n = (h1 - mu) * lax.rsqrt(var + 1e-5) * gamma_ref[...] + beta_ref[...]
    h1a = h1n * 0.5 * (1.0 + lax.erf(h1n * _INV_SQRT2))
    out_ref[...] = jnp.dot(h1a, w2_ref[...],
                           preferred_element_type=jnp.float32) + b2_ref[...]


def _mlp(g, m, b1, gamma, beta, w2, b2):
    eblk = E_TOTAL // _MLP_BLK
    vec = pl.BlockSpec((1, D), lambda j: (0, 0))
    mat = pl.BlockSpec((D, D), lambda j: (0, 0))
    return pl.pallas_call(
        _mlp_body,
        grid=(eblk,),
        in_specs=[
            pl.BlockSpec((_MLP_BLK, D), lambda j: (j, 0)),
            pl.BlockSpec((_MLP_BLK, D), lambda j: (j + eblk, 0)),
            pl.BlockSpec((_MLP_BLK, D), lambda j: (j, 0)),
            vec, vec, vec, mat, vec,
        ],
        out_specs=pl.BlockSpec((_MLP_BLK, D), lambda j: (j, 0)),
        out_shape=jax.ShapeDtypeStruct((E_TOTAL, D), jnp.float32),
    )(g, g, m, b1.reshape(1, D), gamma.reshape(1, D),
      beta.reshape(1, D), w2, b2.reshape(1, D))


def kernel(x, batch, edge_index, edge_rep, num_edges, ptr, W1, b1, gamma,
           beta, W2, b2):
    w_stack = jnp.stack([W1[:D], W1[2 * D:]])
    w1r = W1[D:2 * D]
    tbl = _project_nodes(x, w_stack)
    idx = jnp.concatenate([edge_index[0], edge_index[1] + N_TOTAL])
    g = _sc_gather(tbl, idx)
    m = _rel_proj(edge_rep, w1r)
    out = _mlp(g, m, b1, gamma, beta, W2, b2)
    mask = jnp.ones((BSZ, EDGES), dtype=jnp.float32)
    return out.reshape(BSZ, EDGES, D), mask


# speedup vs baseline: 1.0210x; 1.0210x over previous
import functools
import math

import jax
import jax.numpy as jnp
from jax import lax
from jax.experimental import pallas as pl
from jax.experimental.pallas import tpu as pltpu
from jax.experimental.pallas import tpu_sc as plsc

BSZ, NODES, EDGES, D = 8, 512, 2048, 512
N_TOTAL = BSZ * NODES
E_TOTAL = BSZ * EDGES
G_TOTAL = 2 * E_TOTAL

_PROJ_BLK = 512


def _proj_body(x_ref, w_ref, out_ref):
    out_ref[...] = jnp.dot(x_ref[...], w_ref[0],
                           preferred_element_type=jnp.float32)


def _project_nodes(x16, w_stack16):
    nblk = N_TOTAL // _PROJ_BLK
    return pl.pallas_call(
        _proj_body,
        grid=(2 * nblk,),
        in_specs=[
            pl.BlockSpec((_PROJ_BLK, D), lambda j: (j % nblk, 0)),
            pl.BlockSpec((1, D, D), lambda j: (j // nblk, 0, 0)),
        ],
        out_specs=pl.BlockSpec((_PROJ_BLK, D), lambda j: (j, 0)),
        out_shape=jax.ShapeDtypeStruct((2 * N_TOTAL, D), jnp.float32),
    )(x16, w_stack16)


_NC, _NS = 2, 16
_NW = _NC * _NS
_PER_TILE = G_TOTAL // _NW
_CHUNK = 128
_NCHUNK = _PER_TILE // _CHUNK


@functools.cache
def _sc_gather_kernel():
    mesh = plsc.VectorSubcoreMesh(core_axis_name="c", subcore_axis_name="s",
                                  num_cores=_NC, num_subcores=_NS)

    @functools.partial(
        pl.kernel,
        out_type=jax.ShapeDtypeStruct((G_TOTAL, D), jnp.float32),
        mesh=mesh,
        scratch_types=[
            pltpu.VMEM((_CHUNK,), jnp.int32),
            pltpu.VMEM((_CHUNK, D), jnp.float32),
            pltpu.SemaphoreType.DMA,
        ],
    )
    def body(tbl_hbm, idx_hbm, out_hbm, idx_v, rows_v, sem):
        wid = lax.axis_index("s") * _NC + lax.axis_index("c")
        base = wid * _PER_TILE
        for c in range(_NCHUNK):
            off = base + c * _CHUNK
            pltpu.sync_copy(idx_hbm.at[pl.ds(off, _CHUNK)], idx_v)
            pltpu.async_copy(tbl_hbm.at[idx_v], rows_v, sem).wait()
            pltpu.sync_copy(rows_v, out_hbm.at[pl.ds(off, _CHUNK)])

    return body


def _sc_gather(tbl, idx):
    return _sc_gather_kernel()(tbl, idx)


_MLP_BLK = 1024
_INV_SQRT2 = 1.0 / math.sqrt(2.0)


def _mlp_body(gh_ref, gt_ref, r_ref, w1r_ref, b1_ref, gamma_ref, beta_ref,
              w2_ref, b2_ref, out_ref):
    m = jnp.dot(r_ref[...], w1r_ref[...], preferred_element_type=jnp.float32)
    h1 = m + gh_ref[...] + gt_ref[...] + b1_ref[...]
    mu = jnp.mean(h1, axis=-1, keepdims=True)
    var = jnp.mean((h1 - mu) ** 2, axis=-1, keepdims=True)
    h1n = (h1 - mu) * lax.rsqrt(var + 1e-5) * gamma_ref[...] + beta_ref[...]
    h1a = h1n * 0.5 * (1.0 + lax.erf(h1n * _INV_SQRT2))
    out_ref[...] = jnp.dot(h1a.astype(jnp.bfloat16), w2_ref[...],
                           preferred_element_type=jnp.float32) + b2_ref[...]


def _mlp(g, edge_rep16, w1r16, b1, gamma, beta, w2_16, b2):
    eblk = E_TOTAL // _MLP_BLK
    vec = pl.BlockSpec((1, D), lambda j: (0, 0))
    mat = pl.BlockSpec((D, D), lambda j: (0, 0))
    return pl.pallas_call(
        _mlp_body,
        grid=(eblk,),
        in_specs=[
            pl.BlockSpec((_MLP_BLK, D), lambda j: (j, 0)),
            pl.BlockSpec((_MLP_BLK, D), lambda j: (j + eblk, 0)),
            pl.BlockSpec((_MLP_BLK, D), lambda j: (j, 0)),
            mat, vec, vec, vec, mat, vec,
        ],
        out_specs=pl.BlockSpec((_MLP_BLK, D), lambda j: (j, 0)),
        out_shape=jax.ShapeDtypeStruct((E_TOTAL, D), jnp.float32),
    )(g, g, edge_rep16, w1r16, b1.reshape(1, D), gamma.reshape(1, D),
      beta.reshape(1, D), w2_16, b2.reshape(1, D))


def kernel(x, batch, edge_index, edge_rep, num_edges, ptr, W1, b1, gamma,
           beta, W2, b2):
    w_stack = jnp.stack([W1[:D], W1[2 * D:]])
    w1r = W1[D:2 * D]
    tbl = _project_nodes(x.astype(jnp.bfloat16),
                         w_stack.astype(jnp.bfloat16))
    idx = jnp.concatenate([edge_index[0], edge_index[1] + N_TOTAL])
    g = _sc_gather(tbl, idx)
    out = _mlp(g, edge_rep.astype(jnp.bfloat16), w1r.astype(jnp.bfloat16),
               b1, gamma, beta, W2.astype(jnp.bfloat16), b2)
    mask = jnp.ones((BSZ, EDGES), dtype=jnp.float32)
    return out.reshape(BSZ, EDGES, D), mask


# speedup vs baseline: 1.3708x; 1.3426x over previous
import functools
import math

import jax
import jax.numpy as jnp
from jax import lax
from jax.experimental import pallas as pl
from jax.experimental.pallas import tpu as pltpu
from jax.experimental.pallas import tpu_sc as plsc

BSZ, NODES, EDGES, D = 8, 512, 2048, 512
DH = D // 2
N_TOTAL = BSZ * NODES
E_TOTAL = BSZ * EDGES
G_TOTAL = 2 * E_TOTAL

_PROJ_BLK = 512


def _pack_halves(acc):
    lo = pltpu.bitcast(acc[:, :DH].astype(jnp.bfloat16), jnp.uint16)
    hi = pltpu.bitcast(acc[:, DH:].astype(jnp.bfloat16), jnp.uint16)
    word = (hi.astype(jnp.uint32) << 16) | lo.astype(jnp.uint32)
    return pltpu.bitcast(word, jnp.int32)


def _unpack_halves(word):
    wu = pltpu.bitcast(word, jnp.uint32)
    lo = pltpu.bitcast((wu & 0xFFFF).astype(jnp.uint16), jnp.bfloat16)
    hi = pltpu.bitcast((wu >> 16).astype(jnp.uint16), jnp.bfloat16)
    return lo.astype(jnp.float32), hi.astype(jnp.float32)


def _proj_body(x_ref, w_ref, out_ref):
    acc = jnp.dot(x_ref[...], w_ref[0], preferred_element_type=jnp.float32)
    out_ref[...] = _pack_halves(acc)


def _project_nodes(x, w_stack):
    nblk = N_TOTAL // _PROJ_BLK
    return pl.pallas_call(
        _proj_body,
        grid=(2 * nblk,),
        in_specs=[
            pl.BlockSpec((_PROJ_BLK, D), lambda j: (j % nblk, 0)),
            pl.BlockSpec((1, D, D), lambda j: (j // nblk, 0, 0)),
        ],
        out_specs=pl.BlockSpec((_PROJ_BLK, DH), lambda j: (j, 0)),
        out_shape=jax.ShapeDtypeStruct((2 * N_TOTAL, DH), jnp.int32),
    )(x, w_stack)


_NC, _NS = 2, 16
_NW = _NC * _NS
_PER_TILE = G_TOTAL // _NW
_CHUNK = 128
_NCHUNK = _PER_TILE // _CHUNK


@functools.cache
def _sc_gather_kernel():
    mesh = plsc.VectorSubcoreMesh(core_axis_name="c", subcore_axis_name="s",
                                  num_cores=_NC, num_subcores=_NS)

    @functools.partial(
        pl.kernel,
        out_type=jax.ShapeDtypeStruct((G_TOTAL, DH), jnp.int32),
        mesh=mesh,
        scratch_types=[
            pltpu.VMEM((_CHUNK,), jnp.int32),
            pltpu.VMEM((_CHUNK, DH), jnp.int32),
            pltpu.SemaphoreType.DMA,
        ],
    )
    def body(tbl_hbm, idx_hbm, out_hbm, idx_v, rows_v, sem):
        wid = lax.axis_index("s") * _NC + lax.axis_index("c")
        base = wid * _PER_TILE
        for c in range(_NCHUNK):
            off = base + c * _CHUNK
            pltpu.sync_copy(idx_hbm.at[pl.ds(off, _CHUNK)], idx_v)
            pltpu.async_copy(tbl_hbm.at[idx_v], rows_v, sem).wait()
            pltpu.sync_copy(rows_v, out_hbm.at[pl.ds(off, _CHUNK)])

    return body


def _sc_gather(tbl, idx):
    return _sc_gather_kernel()(tbl, idx)


_MLP_BLK = 1024
_INV_SQRT2 = 1.0 / math.sqrt(2.0)


def _mlp_body(gh_ref, gt_ref, r_ref, w1r_ref, b1_ref, gamma_ref, beta_ref,
              w2_ref, b2_ref, out_ref):
    m = jnp.dot(r_ref[...], w1r_ref[...], preferred_element_type=jnp.float32)
    hl, hh = _unpack_halves(gh_ref[...])
    tl, th = _unpack_halves(gt_ref[...])
    g = jnp.concatenate([hl + tl, hh + th], axis=1)
    h1 = m + g + b1_ref[...]
    mu = jnp.mean(h1, axis=-1, keepdims=True)
    var = jnp.mean((h1 - mu) ** 2, axis=-1, keepdims=True)
    h1n = (h1 - mu) * lax.rsqrt(var + 1e-5) * gamma_ref[...] + beta_ref[...]
    h1a = h1n * 0.5 * (1.0 + lax.erf(h1n * _INV_SQRT2))
    out_ref[...] = jnp.dot(h1a, w2_ref[...],
                           preferred_element_type=jnp.float32) + b2_ref[...]


def _mlp(g, edge_rep, w1r, b1, gamma, beta, w2, b2):
    eblk = E_TOTAL // _MLP_BLK
    vec = pl.BlockSpec((1, D), lambda j: (0, 0))
    mat = pl.BlockSpec((D, D), lambda j: (0, 0))
    return pl.pallas_call(
        _mlp_body,
        grid=(eblk,),
        in_specs=[
            pl.BlockSpec((_MLP_BLK, DH), lambda j: (j, 0)),
            pl.BlockSpec((_MLP_BLK, DH), lambda j: (j + eblk, 0)),
            pl.BlockSpec((_MLP_BLK, D), lambda j: (j, 0)),
            mat, vec, vec, vec, mat, vec,
        ],
        out_specs=pl.BlockSpec((_MLP_BLK, D), lambda j: (j, 0)),
        out_shape=jax.ShapeDtypeStruct((E_TOTAL, D), jnp.float32),
    )(g, g, edge_rep, w1r, b1.reshape(1, D), gamma.reshape(1, D),
      beta.reshape(1, D), w2, b2.reshape(1, D))


def kernel(x, batch, edge_index, edge_rep, num_edges, ptr, W1, b1, gamma,
           beta, W2, b2):
    w_stack = jnp.stack([W1[:D], W1[2 * D:]])
    w1r = W1[D:2 * D]
    tbl = _project_nodes(x, w_stack)
    idx = jnp.concatenate([edge_index[0], edge_index[1] + N_TOTAL])
    g = _sc_gather(tbl, idx)
    out = _mlp(g, edge_rep, w1r, b1, gamma, beta, W2, b2)
    mask = jnp.ones((BSZ, EDGES), dtype=jnp.float32)
    return out.reshape(BSZ, EDGES, D), mask


# speedup vs baseline: 1.5099x; 1.1015x over previous
import functools
import math

import jax
import jax.numpy as jnp
from jax import lax
from jax.experimental import pallas as pl
from jax.experimental.pallas import tpu as pltpu
from jax.experimental.pallas import tpu_sc as plsc

BSZ, NODES, EDGES, D = 8, 512, 2048, 512
DH = D // 2
N_TOTAL = BSZ * NODES
E_TOTAL = BSZ * EDGES
G_TOTAL = 2 * E_TOTAL

_PROJ_BLK = 512


def _pack_halves(acc):
    lo = pltpu.bitcast(acc[:, :DH].astype(jnp.bfloat16), jnp.uint16)
    hi = pltpu.bitcast(acc[:, DH:].astype(jnp.bfloat16), jnp.uint16)
    word = (hi.astype(jnp.uint32) << 16) | lo.astype(jnp.uint32)
    return pltpu.bitcast(word, jnp.int32)


def _unpack_halves(word):
    wu = pltpu.bitcast(word, jnp.uint32)
    lo = pltpu.bitcast((wu & 0xFFFF).astype(jnp.uint16), jnp.bfloat16)
    hi = pltpu.bitcast((wu >> 16).astype(jnp.uint16), jnp.bfloat16)
    return lo.astype(jnp.float32), hi.astype(jnp.float32)


def _proj_body(x_ref, w_ref, out_ref):
    acc = jnp.dot(x_ref[...], w_ref[...], preferred_element_type=jnp.float32)
    out_ref[...] = _pack_halves(acc)


def _project_nodes(x, w1):
    nblk = N_TOTAL // _PROJ_BLK
    return pl.pallas_call(
        _proj_body,
        grid=(2 * nblk,),
        in_specs=[
            pl.BlockSpec((_PROJ_BLK, D), lambda j: (j % nblk, 0)),
            pl.BlockSpec((D, D), lambda j: (2 * (j // nblk), 0)),
        ],
        out_specs=pl.BlockSpec((_PROJ_BLK, DH), lambda j: (j, 0)),
        out_shape=jax.ShapeDtypeStruct((2 * N_TOTAL, DH), jnp.int32),
    )(x, w1)


_NC, _NS = 2, 16
_NW = _NC * _NS
_PER_TILE = G_TOTAL // _NW
_CHUNK = 128
_NCHUNK = _PER_TILE // _CHUNK


@functools.cache
def _sc_gather_kernel():
    mesh = plsc.VectorSubcoreMesh(core_axis_name="c", subcore_axis_name="s",
                                  num_cores=_NC, num_subcores=_NS)

    @functools.partial(
        pl.kernel,
        out_type=jax.ShapeDtypeStruct((G_TOTAL, DH), jnp.int32),
        mesh=mesh,
        scratch_types=[
            pltpu.VMEM((_PER_TILE,), jnp.int32),
            pltpu.VMEM((_CHUNK, DH), jnp.int32),
            pltpu.VMEM((_CHUNK, DH), jnp.int32),
            pltpu.SemaphoreType.DMA,
            pltpu.SemaphoreType.DMA,
        ],
    )
    def body(tbl_hbm, idx_hbm, out_hbm, idx_v, buf0, buf1, gsem, wsem):
        wid = lax.axis_index("s") * _NC + lax.axis_index("c")
        base = wid * _PER_TILE
        bufs = (buf0, buf1)
        pltpu.sync_copy(idx_hbm.at[pl.ds(base, _PER_TILE)], idx_v)
        gcp = {0: pltpu.async_copy(
            tbl_hbm.at[idx_v.at[pl.ds(0, _CHUNK)]], buf0, gsem)}
        wb = {}
        for c in range(_NCHUNK):
            if c + 1 < _NCHUNK:
                if c >= 1:
                    wb[c - 1].wait()
                gcp[c + 1] = pltpu.async_copy(
                    tbl_hbm.at[idx_v.at[pl.ds((c + 1) * _CHUNK, _CHUNK)]],
                    bufs[(c + 1) % 2], gsem)
            gcp[c].wait()
            wb[c] = pltpu.async_copy(
                bufs[c % 2], out_hbm.at[pl.ds(base + c * _CHUNK, _CHUNK)],
                wsem)
        wb[_NCHUNK - 2].wait()
        wb[_NCHUNK - 1].wait()

    return body


def _sc_gather(tbl, idx):
    return _sc_gather_kernel()(tbl, idx)


_MLP_BLK = 1024
_INV_SQRT2 = 1.0 / math.sqrt(2.0)


def _mlp_body(gh_ref, gt_ref, r_ref, w1r_ref, b1_ref, gamma_ref, beta_ref,
              w2_ref, b2_ref, out_ref):
    m = jnp.dot(r_ref[...], w1r_ref[...], preferred_element_type=jnp.float32)
    hl, hh = _unpack_halves(gh_ref[...])
    tl, th = _unpack_halves(gt_ref[...])
    g = jnp.concatenate([hl + tl, hh + th], axis=1)
    h1 = m + g + b1_ref[...]
    mu = jnp.mean(h1, axis=-1, keepdims=True)
    var = jnp.mean((h1 - mu) ** 2, axis=-1, keepdims=True)
    h1n = (h1 - mu) * lax.rsqrt(var + 1e-5) * gamma_ref[...] + beta_ref[...]
    h1a = h1n * 0.5 * (1.0 + lax.erf(h1n * _INV_SQRT2))
    out_ref[...] = jnp.dot(h1a, w2_ref[...],
                           preferred_element_type=jnp.float32) + b2_ref[...]


def _mlp(g, edge_rep, w1, b1, gamma, beta, w2, b2):
    eblk = E_TOTAL // _MLP_BLK
    vec = pl.BlockSpec((1, D), lambda j: (0, 0))
    return pl.pallas_call(
        _mlp_body,
        grid=(eblk,),
        in_specs=[
            pl.BlockSpec((_MLP_BLK, DH), lambda j: (j, 0)),
            pl.BlockSpec((_MLP_BLK, DH), lambda j: (j + eblk, 0)),
            pl.BlockSpec((_MLP_BLK, D), lambda j: (j, 0)),
            pl.BlockSpec((D, D), lambda j: (1, 0)),
            vec, vec, vec,
            pl.BlockSpec((D, D), lambda j: (0, 0)),
            vec,
        ],
        out_specs=pl.BlockSpec((_MLP_BLK, D), lambda j: (j, 0)),
        out_shape=jax.ShapeDtypeStruct((E_TOTAL, D), jnp.float32),
    )(g, g, edge_rep, w1, b1.reshape(1, D), gamma.reshape(1, D),
      beta.reshape(1, D), w2, b2.reshape(1, D))


def kernel(x, batch, edge_index, edge_rep, num_edges, ptr, W1, b1, gamma,
           beta, W2, b2):
    tbl = _project_nodes(x, W1)
    idx = jnp.concatenate([edge_index[0], edge_index[1] + N_TOTAL])
    g = _sc_gather(tbl, idx)
    out = _mlp(g, edge_rep, W1, b1, gamma, beta, W2, b2)
    mask = jnp.ones((BSZ, EDGES), dtype=jnp.float32)
    return out.reshape(BSZ, EDGES, D), mask
